# baseline (speedup 1.0000x reference)
import functools

import jax
import jax.numpy as jnp
from jax import lax
from jax.experimental import pallas as pl
from jax.experimental.pallas import tpu as pltpu
from jax.experimental.pallas import tpu_sc as plsc

D = 64
BBLK = 128
DT = D // 8
NBUF = 4


@functools.lru_cache(maxsize=None)
def _make(S: int, B: int):
    info = plsc.get_sparse_core_info()
    nc = info.num_cores
    nw = nc * info.num_subcores
    assert B == nw * BBLK and S % NBUF == 0
    ngroups = S // NBUF
    mesh = plsc.VectorSubcoreMesh(core_axis_name="c", subcore_axis_name="s")

    @functools.partial(
        pl.kernel,
        mesh=mesh,
        out_type=jax.ShapeDtypeStruct((S, DT, nw, 8, BBLK), jnp.float32),
        scratch_types=[
            pltpu.VMEM((S, BBLK), jnp.int32),
            [pltpu.VMEM((BBLK, D), jnp.float32) for _ in range(NBUF)],
            pltpu.VMEM((DT, 8, BBLK), jnp.float32),
            [pltpu.SemaphoreType.DMA for _ in range(NBUF)],
        ],
        compiler_params=pltpu.CompilerParams(
            use_tc_tiling_on_sc=False, needs_layout_passes=False
        ),
    )
    def body(idsT, tableT, out5, ids_v, gbufs, tbuf, gsems):
        w = lax.axis_index("s") * nc + lax.axis_index("c")
        pltpu.sync_copy(idsT.at[:, pl.ds(w * BBLK, BBLK)], ids_v)

        def start(s, b):
            pltpu.async_copy(tableT.at[ids_v.at[s]], gbufs[b], gsems[b])

        def wait(s, b):
            pltpu.make_async_copy(tableT.at[ids_v.at[s]], gbufs[b], gsems[b]).wait()

        for b in range(NBUF):
            start(b, b)

        iota = lax.iota(jnp.int32, 16)
        rows = [iota + 16 * bb for bb in range(8)]

        def group(j, carry):
            for b in range(NBUF):
                s = j * NBUF + b
                wait(s, b)

                def dline(dt, col):
                    for dd in range(8):
                        c = col + dd
                        for bb in range(8):
                            v = plsc.load_gather(gbufs[b], [rows[bb], c])
                            tbuf[dt, dd, pl.ds(bb * 16, 16)] = v
                    return col + 8

                lax.fori_loop(0, DT, dline, jnp.zeros((16,), jnp.int32))

                pltpu.sync_copy(tbuf, out5.at[s, :, w])

                @pl.when(j < ngroups - 1)
                def _():
                    start(s + NBUF, b)

            return carry

        lax.fori_loop(0, ngroups, group, 0)

    return body


def kernel(input_ids, table):
    b, s = input_ids.shape
    d = table.shape[1]
    idsT = input_ids.T.astype(jnp.int32)
    out5 = _make(s, b)(idsT, table)
    return out5.transpose(2, 4, 0, 1, 3).reshape(b, s, d)


# speedup vs baseline: 1.1957x; 1.1957x over previous
import functools

import jax
import jax.numpy as jnp
from jax import lax
from jax.experimental import pallas as pl
from jax.experimental.pallas import tpu as pltpu
from jax.experimental.pallas import tpu_sc as plsc

D = 64
BBLK = 128
DT = D // 8
NBUF = 4


@functools.lru_cache(maxsize=None)
def _make(S: int, B: int):
    info = plsc.get_sparse_core_info()
    nc = info.num_cores
    nw = nc * info.num_subcores
    assert B == nw * BBLK and S % NBUF == 0
    ngroups = S // NBUF
    mesh = plsc.VectorSubcoreMesh(core_axis_name="c", subcore_axis_name="s")

    @functools.partial(
        pl.kernel,
        mesh=mesh,
        out_type=jax.ShapeDtypeStruct((S, DT, nw, 8, BBLK), jnp.float32),
        scratch_types=[
            pltpu.VMEM((S, BBLK), jnp.int32),
            [pltpu.VMEM((BBLK, D), jnp.float32) for _ in range(NBUF)],
            [pltpu.VMEM((DT, 8, BBLK), jnp.float32) for _ in range(2)],
            [pltpu.SemaphoreType.DMA for _ in range(NBUF)],
            [pltpu.SemaphoreType.DMA for _ in range(2)],
        ],
        compiler_params=pltpu.CompilerParams(
            use_tc_tiling_on_sc=False, needs_layout_passes=False
        ),
    )
    def body(idsT, tableT, out5, ids_v, gbufs, tbufs, gsems, tsems):
        w = lax.axis_index("s") * nc + lax.axis_index("c")
        pltpu.sync_copy(idsT.at[:, pl.ds(w * BBLK, BBLK)], ids_v)

        def start(s, b):
            pltpu.async_copy(tableT.at[ids_v.at[s]], gbufs[b], gsems[b])

        def wait(s, b):
            pltpu.make_async_copy(tableT.at[ids_v.at[s]], gbufs[b], gsems[b]).wait()

        for b in range(NBUF):
            start(b, b)

        iota = lax.iota(jnp.int32, 16)
        rows = [iota + 16 * bb for bb in range(8)]

        def group(j, carry):
            for b in range(NBUF):
                s = j * NBUF + b
                tb = b % 2
                wait(s, b)

                if b >= 2:
                    pltpu.make_async_copy(
                        tbufs[tb], out5.at[s - 2, :, w], tsems[tb]
                    ).wait()
                else:

                    @pl.when(j > 0)
                    def _():
                        pltpu.make_async_copy(
                            tbufs[tb], out5.at[s - 2, :, w], tsems[tb]
                        ).wait()

                def dline(dt, carry2):
                    cb = lax.broadcast(dt * 8, (16,))
                    for dd in range(8):
                        c = cb + dd
                        vs = [
                            plsc.load_gather(gbufs[b], [rows[bb], c])
                            for bb in range(8)
                        ]
                        for bb in range(8):
                            tbufs[tb][dt, dd, pl.ds(bb * 16, 16)] = vs[bb]
                    return carry2

                lax.fori_loop(0, DT, dline, 0)

                pltpu.async_copy(tbufs[tb], out5.at[s, :, w], tsems[tb])

                @pl.when(j < ngroups - 1)
                def _():
                    start(s + NBUF, b)

            return carry

        lax.fori_loop(0, ngroups, group, 0)

        pltpu.make_async_copy(tbufs[0], out5.at[S - 2, :, w], tsems[0]).wait()
        pltpu.make_async_copy(tbufs[1], out5.at[S - 1, :, w], tsems[1]).wait()

    return body


def kernel(input_ids, table):
    b, s = input_ids.shape
    d = table.shape[1]
    idsT = input_ids.T.astype(jnp.int32)
    out5 = _make(s, b)(idsT, table)
    return out5.transpose(2, 4, 0, 1, 3).reshape(b, s, d)


# speedup vs baseline: 2.1645x; 1.8102x over previous
import functools

import jax
import jax.numpy as jnp
from jax import lax
from jax.experimental import pallas as pl
from jax.experimental.pallas import tpu as pltpu
from jax.experimental.pallas import tpu_sc as plsc

D = 64
BBLK = 128
DT = D // 8
NBUF = 4


@functools.lru_cache(maxsize=None)
def _make(S: int, B: int):
    info = plsc.get_sparse_core_info()
    nc = info.num_cores
    nw = nc * info.num_subcores
    assert B == nw * BBLK and S % NBUF == 0
    ngroups = S // NBUF
    mesh = plsc.VectorSubcoreMesh(core_axis_name="c", subcore_axis_name="s")

    @functools.partial(
        pl.kernel,
        mesh=mesh,
        out_type=jax.ShapeDtypeStruct((S, DT, nw, 8, BBLK), jnp.float32),
        scratch_types=[
            pltpu.VMEM((S, BBLK), jnp.int32),
            [pltpu.VMEM((BBLK, D), jnp.float32) for _ in range(NBUF)],
            [pltpu.VMEM((DT, 8, BBLK + 1), jnp.float32) for _ in range(2)],
            [pltpu.SemaphoreType.DMA for _ in range(NBUF)],
            [pltpu.SemaphoreType.DMA for _ in range(2)],
        ],
        compiler_params=pltpu.CompilerParams(
            use_tc_tiling_on_sc=False, needs_layout_passes=False
        ),
    )
    def body(idsT, tableT, out5, ids_v, gbufs, tbufs, gsems, tsems):
        w = lax.axis_index("s") * nc + lax.axis_index("c")
        pltpu.sync_copy(idsT.at[:, pl.ds(w * BBLK, BBLK)], ids_v)

        def start(s, b):
            pltpu.async_copy(tableT.at[ids_v.at[s]], gbufs[b], gsems[b])

        def wait(s, b):
            pltpu.make_async_copy(tableT.at[ids_v.at[s]], gbufs[b], gsems[b]).wait()

        for b in range(NBUF):
            start(b, b)

        iota = lax.iota(jnp.int32, 16)
        nk = D // 16
        dt_idx = [(16 * k + iota) // 8 for k in range(nk)]
        dd_idx = [(16 * k + iota) % 8 for k in range(nk)]

        def group(j, carry):
            for b in range(NBUF):
                s = j * NBUF + b
                tb = b % 2
                wait(s, b)

                if b >= 2:
                    pltpu.make_async_copy(
                        tbufs[tb].at[:, :, pl.ds(0, BBLK)],
                        out5.at[s - 2, :, w],
                        tsems[tb],
                    ).wait()
                else:

                    @pl.when(j > 0)
                    def _():
                        pltpu.make_async_copy(
                            tbufs[tb].at[:, :, pl.ds(0, BBLK)],
                            out5.at[s - 2, :, w],
                            tsems[tb],
                        ).wait()

                def tline(tk, carry2):
                    col = lax.broadcast(tk, (16,))
                    vs = [gbufs[b][tk, pl.ds(16 * k, 16)] for k in range(nk)]
                    for k in range(nk):
                        plsc.store_scatter(
                            tbufs[tb], [dt_idx[k], dd_idx[k], col], vs[k]
                        )
                    return carry2

                lax.fori_loop(0, BBLK, tline, 0)

                pltpu.async_copy(
                    tbufs[tb].at[:, :, pl.ds(0, BBLK)], out5.at[s, :, w], tsems[tb]
                )

                @pl.when(j < ngroups - 1)
                def _():
                    start(s + NBUF, b)

            return carry

        lax.fori_loop(0, ngroups, group, 0)

        pltpu.make_async_copy(
            tbufs[0].at[:, :, pl.ds(0, BBLK)], out5.at[S - 2, :, w], tsems[0]
        ).wait()
        pltpu.make_async_copy(
            tbufs[1].at[:, :, pl.ds(0, BBLK)], out5.at[S - 1, :, w], tsems[1]
        ).wait()

    return body


def kernel(input_ids, table):
    b, s = input_ids.shape
    d = table.shape[1]
    idsT = input_ids.T.astype(jnp.int32)
    out5 = _make(s, b)(idsT, table)
    return out5.transpose(2, 4, 0, 1, 3).reshape(b, s, d)
